# Initial kernel scaffold; baseline (speedup 1.0000x reference)
import jax
import jax.numpy as jnp
from jax.experimental import pallas as pl


def kernel(x, edge_index, W1, b1, W2, b2):
    raise NotImplementedError("write your pallas kernel here")


# speedup vs baseline: 10.2932x; 10.2932x over previous
import functools

import jax
import jax.numpy as jnp
from jax import lax
from jax.experimental import pallas as pl
from jax.experimental.pallas import tpu as pltpu
from jax.experimental.pallas import tpu_sc as plsc

N = 10000
E = 320000
D = 128

NC = 2
NS = 16
DEGW = 16
EPC = E // NC
EPT = EPC // NS
CH = 80
NCHUNK = EPT // CH
N_PAD = 10240
RPT = N_PAD // NS

@functools.cache
def _sc_kernels():
    mesh = plsc.VectorSubcoreMesh(
        core_axis_name="c", subcore_axis_name="s", num_cores=NC, num_subcores=NS
    )
    deg = functools.partial(
        pl.kernel,
        out_type=jax.ShapeDtypeStruct((NC, N_PAD, D), jnp.float32),
        mesh=mesh,
        scratch_types=[
            pltpu.VMEM((CH,), jnp.int32),
            pltpu.VMEM((CH, D), jnp.float32),
            pltpu.VMEM_SHARED((N_PAD, D), jnp.float32),
        ],
    )(_deg_scatter_body)
    edge = functools.partial(
        pl.kernel,
        out_type=jax.ShapeDtypeStruct((NC, N_PAD, D), jnp.float32),
        mesh=mesh,
        scratch_types=[
            pltpu.VMEM((CH,), jnp.int32),
            pltpu.VMEM((CH,), jnp.int32),
            pltpu.VMEM((CH, D), jnp.float32),
            pltpu.SemaphoreType.DMA,
            pltpu.VMEM_SHARED((N_PAD, D), jnp.float32),
        ],
    )(_edge_scatter_body)
    return deg, edge


def _deg_scatter_body(col_hbm, ones_hbm, zeros_hbm, out_hbm, cidx, ones_v, acc):
    c = lax.axis_index("c")
    s = lax.axis_index("s")
    rbase = pl.multiple_of(s * RPT, 8)
    pltpu.sync_copy(zeros_hbm.at[pl.ds(rbase, RPT)], acc.at[pl.ds(rbase, RPT)])
    pltpu.sync_copy(ones_hbm, ones_v)
    plsc.subcore_barrier()
    ebase = c * EPC + s * EPT

    def body(j, carry):
        b = pl.multiple_of(ebase + j * CH, 8)
        pltpu.sync_copy(col_hbm.at[pl.ds(b, CH)], cidx)
        pltpu.sync_copy(ones_v, acc.at[cidx], add=True)
        return carry

    lax.fori_loop(0, NCHUNK, body, 0)
    plsc.subcore_barrier()
    pltpu.sync_copy(acc.at[pl.ds(rbase, RPT)], out_hbm.at[c, pl.ds(rbase, RPT)])


def _edge_scatter_body(g_hbm, row_hbm, col_hbm, zeros_hbm, out_hbm,
                       ridx, cidx, rows, sem, acc):
    c = lax.axis_index("c")
    s = lax.axis_index("s")
    rbase = pl.multiple_of(s * RPT, 8)
    pltpu.sync_copy(zeros_hbm.at[pl.ds(rbase, RPT)], acc.at[pl.ds(rbase, RPT)])
    plsc.subcore_barrier()
    ebase = c * EPC + s * EPT

    def body(j, carry):
        b = pl.multiple_of(ebase + j * CH, 8)
        pltpu.sync_copy(row_hbm.at[pl.ds(b, CH)], ridx)
        pltpu.sync_copy(col_hbm.at[pl.ds(b, CH)], cidx)
        pltpu.async_copy(g_hbm.at[ridx], rows, sem).wait()
        pltpu.sync_copy(rows, acc.at[cidx], add=True)
        return carry

    lax.fori_loop(0, NCHUNK, body, 0)
    plsc.subcore_barrier()
    pltpu.sync_copy(acc.at[pl.ds(rbase, RPT)], out_hbm.at[c, pl.ds(rbase, RPT)])


BLK = 2000


def _layer1_body(deg_ref, x_ref, w_ref, b_ref, g_ref, dis_ref):
    deg = deg_ref[0] + deg_ref[1]
    dis = jnp.where(deg > 0.0, lax.rsqrt(deg), 0.0)
    h = lax.dot_general(
        x_ref[...], w_ref[...], (((1,), (1,)), ((), ())),
        preferred_element_type=jnp.float32,
    ) + b_ref[...]
    g_ref[...] = h * dis[:, :1]
    dis_ref[...] = dis[:, :DEGW]


def _layer1(deg_parts, x, W1, b1):
    return pl.pallas_call(
        _layer1_body,
        grid=(N // BLK,),
        in_specs=[
            pl.BlockSpec((NC, BLK, D), lambda i: (0, i, 0)),
            pl.BlockSpec((BLK, D), lambda i: (i, 0)),
            pl.BlockSpec((D, D), lambda i: (0, 0)),
            pl.BlockSpec((1, D), lambda i: (0, 0)),
        ],
        out_specs=[
            pl.BlockSpec((BLK, D), lambda i: (i, 0)),
            pl.BlockSpec((BLK, DEGW), lambda i: (i, 0)),
        ],
        out_shape=[
            jax.ShapeDtypeStruct((N, D), jnp.float32),
            jax.ShapeDtypeStruct((N, DEGW), jnp.float32),
        ],
    )(deg_parts, x, W1, b1)


def _layer2_body(s_ref, dis_ref, w_ref, b_ref, g_ref):
    s = s_ref[0] + s_ref[1]
    out1 = s * dis_ref[...][:, :1]
    t = jnp.maximum(out1, 0.0)
    h = lax.dot_general(
        t, w_ref[...], (((1,), (1,)), ((), ())),
        preferred_element_type=jnp.float32,
    ) + b_ref[...]
    g_ref[...] = h * dis_ref[...][:, :1]


def _layer2(s_parts, dis, W2, b2):
    return pl.pallas_call(
        _layer2_body,
        grid=(N // BLK,),
        in_specs=[
            pl.BlockSpec((NC, BLK, D), lambda i: (0, i, 0)),
            pl.BlockSpec((BLK, DEGW), lambda i: (i, 0)),
            pl.BlockSpec((D, D), lambda i: (0, 0)),
            pl.BlockSpec((1, D), lambda i: (0, 0)),
        ],
        out_specs=pl.BlockSpec((BLK, D), lambda i: (i, 0)),
        out_shape=jax.ShapeDtypeStruct((N, D), jnp.float32),
    )(s_parts, dis, W2, b2)


def _finish_body(s_ref, dis_ref, out_ref):
    s = s_ref[0] + s_ref[1]
    out_ref[...] = s * dis_ref[...][:, :1]


def _finish(s_parts, dis):
    return pl.pallas_call(
        _finish_body,
        grid=(N // BLK,),
        in_specs=[
            pl.BlockSpec((NC, BLK, D), lambda i: (0, i, 0)),
            pl.BlockSpec((BLK, DEGW), lambda i: (i, 0)),
        ],
        out_specs=pl.BlockSpec((BLK, D), lambda i: (i, 0)),
        out_shape=jax.ShapeDtypeStruct((N, D), jnp.float32),
    )(s_parts, dis)


def kernel(x, edge_index, W1, b1, W2, b2):
    row = edge_index[0].astype(jnp.int32)
    col = edge_index[1].astype(jnp.int32)
    zeros_d = jnp.zeros((N_PAD, D), jnp.float32)
    ones_rows = jnp.ones((CH, D), jnp.float32)

    deg_scatter, edge_scatter = _sc_kernels()
    deg_parts = deg_scatter(col, ones_rows, zeros_d)
    g1, dis = _layer1(deg_parts, x, W1, b1.reshape(1, D))
    s1 = edge_scatter(g1, row, col, zeros_d)
    g2 = _layer2(s1, dis, W2, b2.reshape(1, D))
    s2 = edge_scatter(g2, row, col, zeros_d)
    return _finish(s2, dis)


# speedup vs baseline: 19.0759x; 1.8532x over previous
import functools

import jax
import jax.numpy as jnp
from jax import lax
from jax.experimental import pallas as pl
from jax.experimental.pallas import tpu as pltpu
from jax.experimental.pallas import tpu_sc as plsc

N = 10000
E = 320000
D = 128

NC = 2
NS = 16
DEGW = 16
CH = 128
CPC = E // CH // NC
CPT = CPC // NS
N_PAD = 10240
RPT = N_PAD // NS

@functools.cache
def _sc_kernels():
    mesh = plsc.VectorSubcoreMesh(
        core_axis_name="c", subcore_axis_name="s", num_cores=NC, num_subcores=NS
    )
    deg = functools.partial(
        pl.kernel,
        out_type=jax.ShapeDtypeStruct((NC, N_PAD, D), jnp.float32),
        mesh=mesh,
        scratch_types=[
            pltpu.VMEM((CH,), jnp.int32),
            pltpu.VMEM((CH,), jnp.int32),
            pltpu.VMEM((CH, D), jnp.float32),
            pltpu.SemaphoreType.DMA,
            pltpu.SemaphoreType.DMA,
            pltpu.VMEM_SHARED((N_PAD, D), jnp.float32),
        ],
    )(_deg_scatter_body)
    edge = functools.partial(
        pl.kernel,
        out_type=jax.ShapeDtypeStruct((NC, N_PAD, D), jnp.float32),
        mesh=mesh,
        scratch_types=[
            pltpu.VMEM((CH,), jnp.int32),
            pltpu.VMEM((CH,), jnp.int32),
            pltpu.VMEM((CH,), jnp.int32),
            pltpu.VMEM((CH,), jnp.int32),
            pltpu.VMEM((CH, D), jnp.float32),
            pltpu.VMEM((CH, D), jnp.float32),
            pltpu.SemaphoreType.DMA,
            pltpu.SemaphoreType.DMA,
            pltpu.SemaphoreType.DMA,
            pltpu.SemaphoreType.DMA,
            pltpu.VMEM_SHARED((N_PAD, D), jnp.float32),
        ],
    )(_edge_scatter_body)
    return deg, edge


def _tile_chunks(c, s):
    n = jnp.where(s == NS - 1, CPT + 2, CPT)
    chunk0 = c * CPC + s * CPT
    return chunk0, n


def _deg_scatter_body(col_hbm, ones_hbm, zeros_hbm, out_hbm,
                      cidx0, cidx1, ones_v, sem0, sem1, acc):
    c = lax.axis_index("c")
    s = lax.axis_index("s")
    rbase = pl.multiple_of(s * RPT, 8)
    pltpu.sync_copy(zeros_hbm.at[pl.ds(rbase, RPT)], acc.at[pl.ds(rbase, RPT)])
    pltpu.sync_copy(ones_hbm, ones_v)
    plsc.subcore_barrier()
    chunk0, n = _tile_chunks(c, s)
    m = n // 2

    def _eoff(q):
        return pl.multiple_of((chunk0 + q) * CH, 8)

    pltpu.sync_copy(col_hbm.at[pl.ds(_eoff(0), CH)], cidx0)

    def pair(p, carry):
        qa = 2 * p
        pltpu.async_copy(ones_v, acc.at[cidx0], sem0, add=True)
        pltpu.sync_copy(col_hbm.at[pl.ds(_eoff(qa + 1), CH)], cidx1)
        pltpu.async_copy(ones_v, acc.at[cidx1], sem1, add=True)
        pltpu.make_async_copy(ones_v, acc.at[cidx0], sem0).wait()
        qn = jnp.minimum(qa + 2, n - 1)
        pltpu.sync_copy(col_hbm.at[pl.ds(_eoff(qn), CH)], cidx0)
        pltpu.make_async_copy(ones_v, acc.at[cidx1], sem1).wait()
        return carry

    lax.fori_loop(0, m, pair, 0)
    plsc.subcore_barrier()
    pltpu.sync_copy(acc.at[pl.ds(rbase, RPT)], out_hbm.at[c, pl.ds(rbase, RPT)])


def _edge_scatter_body(g_hbm, row_hbm, col_hbm, zeros_hbm, out_hbm,
                       ridx0, ridx1, cidx0, cidx1, rows0, rows1,
                       semg0, semg1, sems0, sems1, acc):
    c = lax.axis_index("c")
    s = lax.axis_index("s")
    rbase = pl.multiple_of(s * RPT, 8)
    pltpu.sync_copy(zeros_hbm.at[pl.ds(rbase, RPT)], acc.at[pl.ds(rbase, RPT)])
    plsc.subcore_barrier()
    chunk0, n = _tile_chunks(c, s)
    m = n // 2

    def _eoff(q):
        return pl.multiple_of((chunk0 + q) * CH, 8)

    def _load_idx(q, ridx, cidx):
        b = _eoff(q)
        pltpu.sync_copy(row_hbm.at[pl.ds(b, CH)], ridx)
        pltpu.sync_copy(col_hbm.at[pl.ds(b, CH)], cidx)

    _load_idx(0, ridx0, cidx0)
    pltpu.async_copy(g_hbm.at[ridx0], rows0, semg0)
    _load_idx(1, ridx1, cidx1)
    pltpu.async_copy(g_hbm.at[ridx1], rows1, semg1)

    def pair(p, carry):
        qa = 2 * p
        pltpu.make_async_copy(g_hbm.at[ridx0], rows0, semg0).wait()
        pltpu.async_copy(rows0, acc.at[cidx0], sems0, add=True)
        pltpu.make_async_copy(rows0, acc.at[cidx0], sems0).wait()
        _load_idx(jnp.minimum(qa + 2, n - 1), ridx0, cidx0)
        pltpu.async_copy(g_hbm.at[ridx0], rows0, semg0)
        pltpu.make_async_copy(g_hbm.at[ridx1], rows1, semg1).wait()
        pltpu.async_copy(rows1, acc.at[cidx1], sems1, add=True)
        pltpu.make_async_copy(rows1, acc.at[cidx1], sems1).wait()
        _load_idx(jnp.minimum(qa + 3, n - 1), ridx1, cidx1)
        pltpu.async_copy(g_hbm.at[ridx1], rows1, semg1)
        return carry

    lax.fori_loop(0, m, pair, 0)
    pltpu.make_async_copy(g_hbm.at[ridx0], rows0, semg0).wait()
    pltpu.make_async_copy(g_hbm.at[ridx1], rows1, semg1).wait()
    plsc.subcore_barrier()
    pltpu.sync_copy(acc.at[pl.ds(rbase, RPT)], out_hbm.at[c, pl.ds(rbase, RPT)])


BLK = 2000


def _layer1_body(deg_ref, x_ref, w_ref, b_ref, g_ref, dis_ref):
    deg = deg_ref[0] + deg_ref[1]
    dis = jnp.where(deg > 0.0, lax.rsqrt(deg), 0.0)
    h = lax.dot_general(
        x_ref[...], w_ref[...], (((1,), (1,)), ((), ())),
        preferred_element_type=jnp.float32,
    ) + b_ref[...]
    g_ref[...] = h * dis[:, :1]
    dis_ref[...] = dis[:, :DEGW]


def _layer1(deg_parts, x, W1, b1):
    return pl.pallas_call(
        _layer1_body,
        grid=(N // BLK,),
        in_specs=[
            pl.BlockSpec((NC, BLK, D), lambda i: (0, i, 0)),
            pl.BlockSpec((BLK, D), lambda i: (i, 0)),
            pl.BlockSpec((D, D), lambda i: (0, 0)),
            pl.BlockSpec((1, D), lambda i: (0, 0)),
        ],
        out_specs=[
            pl.BlockSpec((BLK, D), lambda i: (i, 0)),
            pl.BlockSpec((BLK, DEGW), lambda i: (i, 0)),
        ],
        out_shape=[
            jax.ShapeDtypeStruct((N, D), jnp.float32),
            jax.ShapeDtypeStruct((N, DEGW), jnp.float32),
        ],
    )(deg_parts, x, W1, b1)


def _layer2_body(s_ref, dis_ref, w_ref, b_ref, g_ref):
    s = s_ref[0] + s_ref[1]
    out1 = s * dis_ref[...][:, :1]
    t = jnp.maximum(out1, 0.0)
    h = lax.dot_general(
        t, w_ref[...], (((1,), (1,)), ((), ())),
        preferred_element_type=jnp.float32,
    ) + b_ref[...]
    g_ref[...] = h * dis_ref[...][:, :1]


def _layer2(s_parts, dis, W2, b2):
    return pl.pallas_call(
        _layer2_body,
        grid=(N // BLK,),
        in_specs=[
            pl.BlockSpec((NC, BLK, D), lambda i: (0, i, 0)),
            pl.BlockSpec((BLK, DEGW), lambda i: (i, 0)),
            pl.BlockSpec((D, D), lambda i: (0, 0)),
            pl.BlockSpec((1, D), lambda i: (0, 0)),
        ],
        out_specs=pl.BlockSpec((BLK, D), lambda i: (i, 0)),
        out_shape=jax.ShapeDtypeStruct((N, D), jnp.float32),
    )(s_parts, dis, W2, b2)


def _finish_body(s_ref, dis_ref, out_ref):
    s = s_ref[0] + s_ref[1]
    out_ref[...] = s * dis_ref[...][:, :1]


def _finish(s_parts, dis):
    return pl.pallas_call(
        _finish_body,
        grid=(N // BLK,),
        in_specs=[
            pl.BlockSpec((NC, BLK, D), lambda i: (0, i, 0)),
            pl.BlockSpec((BLK, DEGW), lambda i: (i, 0)),
        ],
        out_specs=pl.BlockSpec((BLK, D), lambda i: (i, 0)),
        out_shape=jax.ShapeDtypeStruct((N, D), jnp.float32),
    )(s_parts, dis)


def kernel(x, edge_index, W1, b1, W2, b2):
    row = edge_index[0].astype(jnp.int32)
    col = edge_index[1].astype(jnp.int32)
    zeros_d = jnp.zeros((N_PAD, D), jnp.float32)
    ones_rows = jnp.ones((CH, D), jnp.float32)

    deg_scatter, edge_scatter = _sc_kernels()
    deg_parts = deg_scatter(col, ones_rows, zeros_d)
    g1, dis = _layer1(deg_parts, x, W1, b1.reshape(1, D))
    s1 = edge_scatter(g1, row, col, zeros_d)
    g2 = _layer2(s1, dis, W2, b2.reshape(1, D))
    s2 = edge_scatter(g2, row, col, zeros_d)
    return _finish(s2, dis)
